# Initial kernel scaffold; baseline (speedup 1.0000x reference)
import jax
import jax.numpy as jnp
from jax.experimental import pallas as pl


def kernel(x, W):
    raise NotImplementedError("write your pallas kernel here")


# speedup vs baseline: 1.4263x; 1.4263x over previous
import math
import functools

import jax
import jax.numpy as jnp
import numpy as np
from jax import lax
from jax.experimental import pallas as pl
from jax.experimental.pallas import tpu as pltpu
from jax.experimental.pallas import tpu_sc as plsc

_VOCAB = 1000000
_MAX_LEN = 200
_EMB_DIM = 32
_BATCH = 4096

_NC = 2
_NS = 16
_NW = _NC * _NS
_LANES = 16

_TOTAL_ROWS = _BATCH * _MAX_LEN
_ROWS_PER_W = _TOTAL_ROWS // _NW
_SEQ_PER_CHUNK = 8
_CHUNK = _SEQ_PER_CHUNK * _MAX_LEN
_NCHUNKS = _ROWS_PER_W // _CHUNK


def _pe_table() -> np.ndarray:
    dims = np.arange(0, _EMB_DIM, 2, dtype=np.float32)
    freq = np.exp(dims * (-math.log(10000.0) / _EMB_DIM))
    pos = np.arange(0, _MAX_LEN, dtype=np.float32)[:, None]
    pe = np.zeros((_MAX_LEN, _EMB_DIM), dtype=np.float32)
    pe[:, 0::2] = np.sin(pos * freq)
    pe[:, 1::2] = np.cos(pos * freq)
    return pe


_PE_CONST = _pe_table()


def _sc_kernel(x_hbm, w_hbm, pe_hbm, out_hbm, idx_v, rows_v, pe_v, gsem):
    wid = lax.axis_index("s") * _NC + lax.axis_index("c")
    base = wid * _ROWS_PER_W

    pltpu.sync_copy(pe_hbm, pe_v)

    for g in range(_NCHUNKS):
        cbase = base + g * _CHUNK
        pltpu.sync_copy(x_hbm.at[pl.ds(cbase, _CHUNK)], idx_v)
        pltpu.async_copy(w_hbm.at[idx_v], rows_v, gsem).wait()

        def _add_pe(p, carry):
            pe0 = pe_v[p, pl.ds(0, _LANES)]
            pe1 = pe_v[p, pl.ds(_LANES, _LANES)]
            for s in range(_SEQ_PER_CHUNK):
                r = s * _MAX_LEN + p
                rows_v[r, pl.ds(0, _LANES)] += pe0
                rows_v[r, pl.ds(_LANES, _LANES)] += pe1
            return carry

        lax.fori_loop(0, _MAX_LEN, _add_pe, 0, unroll=2)

        pltpu.sync_copy(rows_v, out_hbm.at[pl.ds(cbase, _CHUNK)])


@jax.jit
def _emb_lookup(x_flat, w, pe):
    mesh = plsc.VectorSubcoreMesh(core_axis_name="c", subcore_axis_name="s")
    f = pl.kernel(
        _sc_kernel,
        out_type=jax.ShapeDtypeStruct((_TOTAL_ROWS, _EMB_DIM), jnp.float32),
        mesh=mesh,
        scratch_types=[
            pltpu.VMEM((_CHUNK,), jnp.int32),
            pltpu.VMEM((_CHUNK, _EMB_DIM), jnp.float32),
            pltpu.VMEM((_MAX_LEN, _EMB_DIM), jnp.float32),
            pltpu.SemaphoreType.DMA,
        ],
        compiler_params=pltpu.CompilerParams(use_tc_tiling_on_sc=False),
    )
    return f(x_flat, w, pe)


def kernel(x, W):
    x_flat = x.reshape(_TOTAL_ROWS).astype(jnp.int32)
    pe = jnp.asarray(_PE_CONST)
    out = _emb_lookup(x_flat, W, pe)
    return out.reshape(_BATCH, _MAX_LEN, _EMB_DIM)


# speedup vs baseline: 1.4803x; 1.0379x over previous
import math
import functools

import jax
import jax.numpy as jnp
import numpy as np
from jax import lax
from jax.experimental import pallas as pl
from jax.experimental.pallas import tpu as pltpu
from jax.experimental.pallas import tpu_sc as plsc

_VOCAB = 1000000
_MAX_LEN = 200
_EMB_DIM = 32
_BATCH = 4096

_NC = 2
_NS = 16
_NW = _NC * _NS
_LANES = 16

_QSPLIT = 4
_QB = _BATCH // _QSPLIT
_UNITS = _MAX_LEN * _QSPLIT
_UNITS_PER_W = _UNITS // _NW


def _pe_table() -> np.ndarray:
    dims = np.arange(0, _EMB_DIM, 2, dtype=np.float32)
    freq = np.exp(dims * (-math.log(10000.0) / _EMB_DIM))
    pos = np.arange(0, _MAX_LEN, dtype=np.float32)[:, None]
    pe = np.zeros((_MAX_LEN, _EMB_DIM), dtype=np.float32)
    pe[:, 0::2] = np.sin(pos * freq)
    pe[:, 1::2] = np.cos(pos * freq)
    return pe


_PE_CONST = _pe_table()


def _sc_kernel(xt_hbm, w_hbm, pe_hbm, out_hbm, idx_v, rows_v, pe_v, gsem):
    wid = lax.axis_index("s") * _NC + lax.axis_index("c")

    pltpu.sync_copy(pe_hbm, pe_v)

    for j in range(_UNITS_PER_W):
        u = wid * _UNITS_PER_W + j
        l = u // _QSPLIT
        q = u % _QSPLIT

        pltpu.sync_copy(xt_hbm.at[l, pl.ds(q * _QB, _QB)], idx_v)
        pltpu.async_copy(w_hbm.at[idx_v], rows_v, gsem).wait()

        pe0 = pe_v[l, pl.ds(0, _LANES)]
        pe1 = pe_v[l, pl.ds(_LANES, _LANES)]

        def _add_pe(r, carry):
            rows_v[r, pl.ds(0, _LANES)] += pe0
            rows_v[r, pl.ds(_LANES, _LANES)] += pe1
            return carry

        lax.fori_loop(0, _QB, _add_pe, 0, unroll=4)

        pltpu.sync_copy(rows_v, out_hbm.at[l, pl.ds(q * _QB, _QB), :])


@jax.jit
def _emb_lookup(x_t, w, pe):
    mesh = plsc.VectorSubcoreMesh(core_axis_name="c", subcore_axis_name="s")
    f = pl.kernel(
        _sc_kernel,
        out_type=jax.ShapeDtypeStruct((_MAX_LEN, _BATCH, _EMB_DIM), jnp.float32),
        mesh=mesh,
        scratch_types=[
            pltpu.VMEM((_QB,), jnp.int32),
            pltpu.VMEM((_QB, _EMB_DIM), jnp.float32),
            pltpu.VMEM((_MAX_LEN, _EMB_DIM), jnp.float32),
            pltpu.SemaphoreType.DMA,
        ],
        compiler_params=pltpu.CompilerParams(use_tc_tiling_on_sc=False),
    )
    return f(x_t, w, pe)


def kernel(x, W):
    x_t = x.T.astype(jnp.int32)
    pe = jnp.asarray(_PE_CONST)
    out = _emb_lookup(x_t, W, pe)
    return out.transpose(1, 0, 2)
